# Initial kernel scaffold; baseline (speedup 1.0000x reference)
import jax
import jax.numpy as jnp
from jax.experimental import pallas as pl


def kernel(data, prev_features, conv_w, conv_b):
    raise NotImplementedError("write your pallas kernel here")


# speedup vs baseline: 16.7088x; 16.7088x over previous
import functools
import jax
import jax.numpy as jnp
from jax import lax
from jax.experimental import pallas as pl
from jax.experimental.pallas import tpu as pltpu
from jax.experimental.pallas import tpu_sc as plsc

N = 128
C = 4
CP = 4
P = 1500
PP = 1536
PR = 12
GX = 124
NV = GX * 128 * 128
ZCHUNK = NV // 16
SROWS = (8 * PP) // 128
SROWS_PAD = 128
GROWS = (64 * PP) // 128
SPT = SROWS_PAD // 16
GPT = GROWS // 16

_IJK = [(i, j, k) for i in range(2) for j in range(2) for k in range(2)]
_UVW = [(u, v, w) for u in range(4) for v in range(4) for w in range(4)]


def _prep_body(cells_ref, corners_ref, data_ref, feat_ref, mask_ref,
               sidx_ref, svals_ref, gidx_ref, d8_ref):
    xc = cells_ref[0]
    yc = cells_ref[1]
    zc = cells_ref[2]
    dx = data_ref[0]
    dy = data_ref[1]
    dz = data_ref[2]
    m = mask_ref[0]
    dlist = []
    for t, (i, j, k) in enumerate(_IJK):
        ax = corners_ref[i]
        ay = corners_ref[2 + j]
        az = corners_ref[4 + k]
        d = jnp.sqrt((dx - ax) ** 2 + (dy - ay) ** 2 + (dz - az) ** 2)
        dlist.append(d)
        sidx_ref[t] = jnp.clip(
            ((xc + (i - 3)) * 128 + (yc + j)) * 128 + (zc + k), 0, NV - 1)
    for ch in range(CP):
        f = feat_ref[ch] * m
        for t in range(8):
            svals_ref[ch, t] = f / dlist[t]
    for t, (i, j, k) in enumerate(_IJK):
        d8_ref[t] = dlist[4 * i + 2 * j * k] * m
    for t2, (u, v, w) in enumerate(_UVW):
        gidx_ref[t2] = jnp.clip(
            ((xc + (u - 3)) * 128 + (yc + v)) * 128 + (zc + w), 0, NV - 1)


def _final_body(g_ref, w_ref, d_ref, b_ref, o_ref):
    y = jnp.dot(g_ref[...], w_ref[...], preferred_element_type=jnp.float32)
    y = y + b_ref[...]
    d8 = d_ref[...]
    cols = [jnp.sum(y[:, c * 8:(c + 1) * 8] * d8, axis=1, keepdims=True)
            for c in range(C)]
    cols.append(jnp.zeros((PP, 8 - C), jnp.float32))
    o_ref[...] = jnp.concatenate(cols, axis=1) * 0.125


def _sc_body(sidx_hbm, svals_hbm, gidx_hbm, zeros_hbm, out_hbm,
             grid_sh, ib, vb, sem):
    cid = lax.axis_index("c")
    sid = lax.axis_index("s")
    zbase = pl.multiple_of(sid * ZCHUNK, 8)
    sbase = pl.multiple_of(sid * SPT, 8)
    for r in range(2):
        ch = cid * 2 + r
        pltpu.sync_copy(zeros_hbm.at[pl.ds(zbase, ZCHUNK)],
                        grid_sh.at[pl.ds(zbase, ZCHUNK)])
        plsc.subcore_barrier()
        pltpu.sync_copy(sidx_hbm.at[pl.ds(sbase, SPT)], ib)
        pltpu.sync_copy(svals_hbm.at[ch, pl.ds(sbase, SPT)], vb)
        for j in range(SPT):
            pltpu.sync_copy(vb.at[j], grid_sh.at[ib.at[j]], add=True)
        plsc.subcore_barrier()

        def _gather(g, carry):
            gbase = pl.multiple_of(sid * GPT + g * SPT, 8)
            pltpu.sync_copy(gidx_hbm.at[pl.ds(gbase, SPT)], ib)
            for j in range(SPT):
                pltpu.async_copy(grid_sh.at[ib.at[j]], vb.at[j], sem).wait()
            pltpu.sync_copy(vb, out_hbm.at[ch, pl.ds(gbase, SPT)])
            return carry

        lax.fori_loop(0, GPT // SPT, _gather, 0)
        plsc.subcore_barrier()


@functools.lru_cache(maxsize=1)
def _sc_splat_gather():
    return functools.partial(
        pl.kernel,
        mesh=plsc.VectorSubcoreMesh(core_axis_name="c", subcore_axis_name="s"),
        out_type=jax.ShapeDtypeStruct((C, GROWS, 128), jnp.float32),
        scratch_types=[
            pltpu.VMEM_SHARED((NV,), jnp.float32),
            pltpu.VMEM((SPT, 128), jnp.int32),
            pltpu.VMEM((SPT, 128), jnp.float32),
            pltpu.SemaphoreType.DMA,
        ],
    )(_sc_body)


def _pad_points(a):
    return jnp.concatenate([a, jnp.broadcast_to(a[0], (PP - P,) + a.shape[1:])])


def kernel(data, prev_features, conv_w, conv_b):
    feat = prev_features.reshape(CP, P)
    maxd = jnp.ceil(jnp.max(data, axis=0))
    mind = jnp.floor(jnp.min(data, axis=0))
    s = (maxd - mind) / (N - 9)
    xs = jnp.linspace(mind[0] - s[0] * 4, maxd[0] + s[0] * 4, N)
    ys = jnp.linspace(mind[1] - s[1] * 4, maxd[1] + s[1] * 4, N)
    zs = jnp.linspace(mind[2] - s[2] * 4, maxd[2] + s[2] * 4, N)
    xc = lax.stop_gradient(jnp.floor((data[:, 0] - xs[0]) / s[0])).astype(jnp.int32)
    yc = lax.stop_gradient(jnp.floor((data[:, 1] - ys[0]) / s[1])).astype(jnp.int32)
    zc = lax.stop_gradient(jnp.floor((data[:, 2] - zs[2 - 2]) / s[2])).astype(jnp.int32)
    corners = jnp.stack([
        xs[xc], xs[xc + 1], ys[yc], ys[yc + 1], zs[zc], zs[zc + 1]
    ]).astype(jnp.int32).astype(jnp.float32)

    cells = jnp.stack([xc, yc, zc])
    mask = (jnp.arange(PP) < P).astype(jnp.float32).reshape(1, PR, 128)
    cells_p = _pad_points(cells.T).T.reshape(3, PR, 128)
    corners_p = _pad_points(corners.T).T.reshape(6, PR, 128)
    data_p = _pad_points(data).T.reshape(3, PR, 128)
    feat_p = _pad_points(feat.T).T.reshape(CP, PR, 128)

    sidx, svals, gidx, d8 = pl.pallas_call(
        _prep_body,
        out_shape=[
            jax.ShapeDtypeStruct((8, PR, 128), jnp.int32),
            jax.ShapeDtypeStruct((CP, 8, PR, 128), jnp.float32),
            jax.ShapeDtypeStruct((64, PR, 128), jnp.int32),
            jax.ShapeDtypeStruct((8, PR, 128), jnp.float32),
        ],
    )(cells_p, corners_p, data_p, feat_p, mask)

    pad_rows = SROWS_PAD - SROWS
    G = _sc_splat_gather()(
        jnp.concatenate([sidx.reshape(SROWS, 128),
                         jnp.zeros((pad_rows, 128), jnp.int32)]),
        jnp.concatenate([svals.reshape(CP, SROWS, 128),
                         jnp.zeros((CP, pad_rows, 128), jnp.float32)], axis=1),
        gidx.reshape(GROWS, 128),
        jnp.zeros((NV,), jnp.float32),
    )
    Gm = G.reshape(CP, 64, PP).transpose(2, 0, 1).reshape(PP, CP * 64)

    wp = jnp.pad(conv_w, ((0, 0), (0, 0), (1, 1), (1, 1), (1, 1)))
    blocks = [wp[:, :, 1 - i:5 - i, 1 - j:5 - j, 1 - k:5 - k]
              for (i, j, k) in _IJK]
    Wm = jnp.stack(blocks, 1).reshape(C, 8, CP, 64)
    Wm = Wm.transpose(2, 3, 0, 1).reshape(CP * 64, C * 8)
    brow = jnp.repeat(conv_b, 8).reshape(1, C * 8)
    D8m = d8.reshape(8, PP).T

    out = pl.pallas_call(
        _final_body,
        out_shape=jax.ShapeDtypeStruct((PP, 8), jnp.float32),
    )(Gm, Wm, D8m, brow)
    return out[:P, :C].T.reshape(1, C, P, 1, 1)


# speedup vs baseline: 17.5863x; 1.0525x over previous
import functools
import jax
import jax.numpy as jnp
from jax import lax
from jax.experimental import pallas as pl
from jax.experimental.pallas import tpu as pltpu
from jax.experimental.pallas import tpu_sc as plsc

N = 128
C = 4
CP = 4
P = 1500
PP = 1536
PR = 12
GX = 124
NV = GX * 128 * 128
ZCHUNK = NV // 16
SROWS = (8 * PP) // 128
SROWS_PAD = 128
GROWS = (64 * PP) // 128
SPT = SROWS_PAD // 16
GPT = GROWS // 16

_IJK = [(i, j, k) for i in range(2) for j in range(2) for k in range(2)]
_UVW = [(u, v, w) for u in range(4) for v in range(4) for w in range(4)]


def _prep_body(cells_ref, corners_ref, data_ref, feat_ref, mask_ref,
               sidx_ref, svals_ref, gidx_ref, d8_ref):
    xc = cells_ref[0]
    yc = cells_ref[1]
    zc = cells_ref[2]
    dx = data_ref[0]
    dy = data_ref[1]
    dz = data_ref[2]
    m = mask_ref[0]
    dlist = []
    for t, (i, j, k) in enumerate(_IJK):
        ax = corners_ref[i]
        ay = corners_ref[2 + j]
        az = corners_ref[4 + k]
        d = jnp.sqrt((dx - ax) ** 2 + (dy - ay) ** 2 + (dz - az) ** 2)
        dlist.append(d)
        sidx_ref[t] = jnp.clip(
            ((xc + (i - 3)) * 128 + (yc + j)) * 128 + (zc + k), 0, NV - 1)
    for ch in range(CP):
        f = feat_ref[ch] * m
        for t in range(8):
            svals_ref[ch, t] = f / dlist[t]
    for t, (i, j, k) in enumerate(_IJK):
        d8_ref[t] = dlist[4 * i + 2 * j * k] * m
    for t2, (u, v, w) in enumerate(_UVW):
        gidx_ref[t2] = jnp.clip(
            ((xc + (u - 3)) * 128 + (yc + v)) * 128 + (zc + w), 0, NV - 1)


def _final_body(g_ref, w_ref, d_ref, b_ref, o_ref):
    y = jnp.dot(g_ref[...], w_ref[...], preferred_element_type=jnp.float32)
    y = y + b_ref[...]
    d8 = d_ref[...]
    cols = [jnp.sum(y[:, c * 8:(c + 1) * 8] * d8, axis=1, keepdims=True)
            for c in range(C)]
    cols.append(jnp.zeros((PP, 8 - C), jnp.float32))
    o_ref[...] = jnp.concatenate(cols, axis=1) * 0.125


def _sc_body(sidx_hbm, svals_hbm, gidx_hbm, zeros_hbm, out_hbm,
             grid_sh, ib, vb, sem):
    cid = lax.axis_index("c")
    sid = lax.axis_index("s")
    zbase = pl.multiple_of(sid * ZCHUNK, 8)
    sbase = pl.multiple_of(sid * SPT, 8)
    for r in range(2):
        ch = cid * 2 + r
        pltpu.sync_copy(zeros_hbm.at[pl.ds(zbase, ZCHUNK)],
                        grid_sh.at[pl.ds(zbase, ZCHUNK)])
        plsc.subcore_barrier()
        pltpu.sync_copy(sidx_hbm.at[pl.ds(sbase, SPT)], ib)
        pltpu.sync_copy(svals_hbm.at[ch, pl.ds(sbase, SPT)], vb)
        for j in range(SPT):
            pltpu.sync_copy(vb.at[j], grid_sh.at[ib.at[j]], add=True)
        plsc.subcore_barrier()

        def _gather(g, carry):
            gbase = pl.multiple_of(sid * GPT + g * SPT, 8)
            pltpu.sync_copy(gidx_hbm.at[pl.ds(gbase, SPT)], ib)
            cps = [pltpu.async_copy(grid_sh.at[ib.at[j]], vb.at[j], sem)
                   for j in range(SPT)]
            for cp in cps:
                cp.wait()
            pltpu.sync_copy(vb, out_hbm.at[ch, pl.ds(gbase, SPT)])
            return carry

        lax.fori_loop(0, GPT // SPT, _gather, 0)
        plsc.subcore_barrier()


@functools.lru_cache(maxsize=1)
def _sc_splat_gather():
    return functools.partial(
        pl.kernel,
        mesh=plsc.VectorSubcoreMesh(core_axis_name="c", subcore_axis_name="s"),
        out_type=jax.ShapeDtypeStruct((C, GROWS, 128), jnp.float32),
        scratch_types=[
            pltpu.VMEM_SHARED((NV,), jnp.float32),
            pltpu.VMEM((SPT, 128), jnp.int32),
            pltpu.VMEM((SPT, 128), jnp.float32),
            pltpu.SemaphoreType.DMA,
        ],
    )(_sc_body)


def _pad_points(a):
    return jnp.concatenate([a, jnp.broadcast_to(a[0], (PP - P,) + a.shape[1:])])


def kernel(data, prev_features, conv_w, conv_b):
    feat = prev_features.reshape(CP, P)
    maxd = jnp.ceil(jnp.max(data, axis=0))
    mind = jnp.floor(jnp.min(data, axis=0))
    s = (maxd - mind) / (N - 9)
    xs = jnp.linspace(mind[0] - s[0] * 4, maxd[0] + s[0] * 4, N)
    ys = jnp.linspace(mind[1] - s[1] * 4, maxd[1] + s[1] * 4, N)
    zs = jnp.linspace(mind[2] - s[2] * 4, maxd[2] + s[2] * 4, N)
    xc = lax.stop_gradient(jnp.floor((data[:, 0] - xs[0]) / s[0])).astype(jnp.int32)
    yc = lax.stop_gradient(jnp.floor((data[:, 1] - ys[0]) / s[1])).astype(jnp.int32)
    zc = lax.stop_gradient(jnp.floor((data[:, 2] - zs[2 - 2]) / s[2])).astype(jnp.int32)
    corners = jnp.stack([
        xs[xc], xs[xc + 1], ys[yc], ys[yc + 1], zs[zc], zs[zc + 1]
    ]).astype(jnp.int32).astype(jnp.float32)

    cells = jnp.stack([xc, yc, zc])
    mask = (jnp.arange(PP) < P).astype(jnp.float32).reshape(1, PR, 128)
    cells_p = _pad_points(cells.T).T.reshape(3, PR, 128)
    corners_p = _pad_points(corners.T).T.reshape(6, PR, 128)
    data_p = _pad_points(data).T.reshape(3, PR, 128)
    feat_p = _pad_points(feat.T).T.reshape(CP, PR, 128)

    sidx, svals, gidx, d8 = pl.pallas_call(
        _prep_body,
        out_shape=[
            jax.ShapeDtypeStruct((8, PR, 128), jnp.int32),
            jax.ShapeDtypeStruct((CP, 8, PR, 128), jnp.float32),
            jax.ShapeDtypeStruct((64, PR, 128), jnp.int32),
            jax.ShapeDtypeStruct((8, PR, 128), jnp.float32),
        ],
    )(cells_p, corners_p, data_p, feat_p, mask)

    pad_rows = SROWS_PAD - SROWS
    G = _sc_splat_gather()(
        jnp.concatenate([sidx.reshape(SROWS, 128),
                         jnp.zeros((pad_rows, 128), jnp.int32)]),
        jnp.concatenate([svals.reshape(CP, SROWS, 128),
                         jnp.zeros((CP, pad_rows, 128), jnp.float32)], axis=1),
        gidx.reshape(GROWS, 128),
        jnp.zeros((NV,), jnp.float32),
    )
    Gm = G.reshape(CP, 64, PP).transpose(2, 0, 1).reshape(PP, CP * 64)

    wp = jnp.pad(conv_w, ((0, 0), (0, 0), (1, 1), (1, 1), (1, 1)))
    blocks = [wp[:, :, 1 - i:5 - i, 1 - j:5 - j, 1 - k:5 - k]
              for (i, j, k) in _IJK]
    Wm = jnp.stack(blocks, 1).reshape(C, 8, CP, 64)
    Wm = Wm.transpose(2, 3, 0, 1).reshape(CP * 64, C * 8)
    brow = jnp.repeat(conv_b, 8).reshape(1, C * 8)
    D8m = d8.reshape(8, PP).T

    out = pl.pallas_call(
        _final_body,
        out_shape=jax.ShapeDtypeStruct((PP, 8), jnp.float32),
    )(Gm, Wm, D8m, brow)
    return out[:P, :C].T.reshape(1, C, P, 1, 1)


# speedup vs baseline: 18.6551x; 1.0608x over previous
import functools
import jax
import jax.numpy as jnp
from jax import lax
from jax.experimental import pallas as pl
from jax.experimental.pallas import tpu as pltpu
from jax.experimental.pallas import tpu_sc as plsc

N = 128
C = 4
CP = 4
P = 1500
PP = 1536
PR = 12
GX = 124
NV = GX * 128 * 128
ZCHUNK = NV // 16
SROWS = (8 * PP) // 128
SROWS_PAD = 128
GROWS = (64 * PP) // 128
SPT = SROWS_PAD // 16
GPT = GROWS // 16

_IJK = [(i, j, k) for i in range(2) for j in range(2) for k in range(2)]
_UVW = [(u, v, w) for u in range(4) for v in range(4) for w in range(4)]


def _prep_body(cells_ref, corners_ref, data_ref, feat_ref, mask_ref,
               sidx_ref, svals_ref, gidx_ref, d8_ref):
    xc = cells_ref[0]
    yc = cells_ref[1]
    zc = cells_ref[2]
    dx = data_ref[0]
    dy = data_ref[1]
    dz = data_ref[2]
    m = mask_ref[0]
    dlist = []
    for t, (i, j, k) in enumerate(_IJK):
        ax = corners_ref[i]
        ay = corners_ref[2 + j]
        az = corners_ref[4 + k]
        d = jnp.sqrt((dx - ax) ** 2 + (dy - ay) ** 2 + (dz - az) ** 2)
        dlist.append(d)
        sidx_ref[t] = jnp.clip(
            ((xc + (i - 3)) * 128 + (yc + j)) * 128 + (zc + k), 0, NV - 1)
    for ch in range(CP):
        f = feat_ref[ch] * m
        for t in range(8):
            svals_ref[ch, t] = f / dlist[t]
    for t, (i, j, k) in enumerate(_IJK):
        d8_ref[t] = dlist[4 * i + 2 * j * k] * m
    for t2, (u, v, w) in enumerate(_UVW):
        gidx_ref[t2] = jnp.clip(
            ((xc + (u - 3)) * 128 + (yc + v)) * 128 + (zc + w), 0, NV - 1)


def _final_body(g_ref, w_ref, d_ref, b_ref, o_ref):
    y = lax.dot_general(w_ref[...], g_ref[...], (((0,), (0,)), ((), ())),
                        preferred_element_type=jnp.float32)
    y = y + b_ref[...]
    d8 = d_ref[...]
    rows = [jnp.sum(y[c * 8:(c + 1) * 8, :] * d8, axis=0, keepdims=True)
            for c in range(C)]
    o_ref[...] = jnp.concatenate(rows, axis=0) * 0.125


def _sc_body(sidx_hbm, svals_hbm, gidx_hbm, zeros_hbm, out_hbm,
             grid_sh, ib, vb, sem):
    cid = lax.axis_index("c")
    sid = lax.axis_index("s")
    zbase = pl.multiple_of(sid * ZCHUNK, 8)
    sbase = pl.multiple_of(sid * SPT, 8)
    for r in range(2):
        ch = cid * 2 + r
        zcp = pltpu.async_copy(zeros_hbm, grid_sh.at[pl.ds(zbase, ZCHUNK)],
                               sem)
        pltpu.sync_copy(sidx_hbm.at[pl.ds(sbase, SPT)], ib)
        pltpu.sync_copy(svals_hbm.at[ch, pl.ds(sbase, SPT)], vb)
        zcp.wait()
        plsc.subcore_barrier()
        for j in range(SPT):
            pltpu.sync_copy(vb.at[j], grid_sh.at[ib.at[j]], add=True)
        plsc.subcore_barrier()

        def _gather(g, carry):
            gbase = pl.multiple_of(sid * GPT + g * SPT, 8)
            pltpu.sync_copy(gidx_hbm.at[pl.ds(gbase, SPT)], ib)
            cps = [pltpu.async_copy(grid_sh.at[ib.at[j]], vb.at[j], sem)
                   for j in range(SPT)]
            for cp in cps:
                cp.wait()
            pltpu.sync_copy(vb, out_hbm.at[ch, pl.ds(gbase, SPT)])
            return carry

        lax.fori_loop(0, GPT // SPT, _gather, 0)
        plsc.subcore_barrier()


@functools.lru_cache(maxsize=1)
def _sc_splat_gather():
    return functools.partial(
        pl.kernel,
        mesh=plsc.VectorSubcoreMesh(core_axis_name="c", subcore_axis_name="s"),
        out_type=jax.ShapeDtypeStruct((C, GROWS, 128), jnp.float32),
        scratch_types=[
            pltpu.VMEM_SHARED((NV,), jnp.float32),
            pltpu.VMEM((SPT, 128), jnp.int32),
            pltpu.VMEM((SPT, 128), jnp.float32),
            pltpu.SemaphoreType.DMA,
        ],
    )(_sc_body)


def _pad_points(a):
    return jnp.concatenate([a, jnp.broadcast_to(a[0], (PP - P,) + a.shape[1:])])


def kernel(data, prev_features, conv_w, conv_b):
    feat = prev_features.reshape(CP, P)
    maxd = jnp.ceil(jnp.max(data, axis=0))
    mind = jnp.floor(jnp.min(data, axis=0))
    s = (maxd - mind) / (N - 9)
    xs = jnp.linspace(mind[0] - s[0] * 4, maxd[0] + s[0] * 4, N)
    ys = jnp.linspace(mind[1] - s[1] * 4, maxd[1] + s[1] * 4, N)
    zs = jnp.linspace(mind[2] - s[2] * 4, maxd[2] + s[2] * 4, N)
    xc = lax.stop_gradient(jnp.floor((data[:, 0] - xs[0]) / s[0])).astype(jnp.int32)
    yc = lax.stop_gradient(jnp.floor((data[:, 1] - ys[0]) / s[1])).astype(jnp.int32)
    zc = lax.stop_gradient(jnp.floor((data[:, 2] - zs[2 - 2]) / s[2])).astype(jnp.int32)
    corners = jnp.stack([
        xs[xc], xs[xc + 1], ys[yc], ys[yc + 1], zs[zc], zs[zc + 1]
    ]).astype(jnp.int32).astype(jnp.float32)

    cells = jnp.stack([xc, yc, zc])
    mask = (jnp.arange(PP) < P).astype(jnp.float32).reshape(1, PR, 128)
    cells_p = _pad_points(cells.T).T.reshape(3, PR, 128)
    corners_p = _pad_points(corners.T).T.reshape(6, PR, 128)
    data_p = _pad_points(data).T.reshape(3, PR, 128)
    feat_p = _pad_points(feat.T).T.reshape(CP, PR, 128)

    sidx, svals, gidx, d8 = pl.pallas_call(
        _prep_body,
        out_shape=[
            jax.ShapeDtypeStruct((8, PR, 128), jnp.int32),
            jax.ShapeDtypeStruct((CP, 8, PR, 128), jnp.float32),
            jax.ShapeDtypeStruct((64, PR, 128), jnp.int32),
            jax.ShapeDtypeStruct((8, PR, 128), jnp.float32),
        ],
    )(cells_p, corners_p, data_p, feat_p, mask)

    pad_rows = SROWS_PAD - SROWS
    G = _sc_splat_gather()(
        jnp.concatenate([sidx.reshape(SROWS, 128),
                         jnp.zeros((pad_rows, 128), jnp.int32)]),
        jnp.concatenate([svals.reshape(CP, SROWS, 128),
                         jnp.zeros((CP, pad_rows, 128), jnp.float32)], axis=1),
        gidx.reshape(GROWS, 128),
        jnp.zeros((ZCHUNK,), jnp.float32),
    )
    Gm = G.reshape(CP * 64, PP)

    wp = jnp.pad(conv_w, ((0, 0), (0, 0), (1, 1), (1, 1), (1, 1)))
    blocks = [wp[:, :, 1 - i:5 - i, 1 - j:5 - j, 1 - k:5 - k]
              for (i, j, k) in _IJK]
    Wm = jnp.stack(blocks, 1).reshape(C, 8, CP, 64)
    Wm = Wm.transpose(2, 3, 0, 1).reshape(CP * 64, C * 8)
    bcol = jnp.repeat(conv_b, 8).reshape(C * 8, 1)
    D8m = d8.reshape(8, PP)

    out = pl.pallas_call(
        _final_body,
        out_shape=jax.ShapeDtypeStruct((C, PP), jnp.float32),
    )(Gm, Wm, D8m, bcol)
    return out[:, :P].reshape(1, C, P, 1, 1)


# speedup vs baseline: 19.5596x; 1.0485x over previous
import functools
import jax
import jax.numpy as jnp
from jax import lax
from jax.experimental import pallas as pl
from jax.experimental.pallas import tpu as pltpu
from jax.experimental.pallas import tpu_sc as plsc

N = 128
C = 4
CP = 4
P = 1500
PP = 1536
PR = 12
GD = 124
NV = GD * GD * GD
NVP = 1906688
ZCHUNK = NVP // 16
GCH = 24
SROWS = (8 * PP) // 128
SROWS_PAD = 128
GROWS = (64 * PP) // 128
SPT = SROWS_PAD // 16
GPT = GROWS // 16

_IJK = [(i, j, k) for i in range(2) for j in range(2) for k in range(2)]
_UVW = [(u, v, w) for u in range(4) for v in range(4) for w in range(4)]


def _prep_body(cells_ref, corners_ref, data_ref, feat_ref, mask_ref,
               sidx_ref, svals_ref, gidx_ref, d8_ref):
    xc = cells_ref[0]
    yc = cells_ref[1]
    zc = cells_ref[2]
    dx = data_ref[0]
    dy = data_ref[1]
    dz = data_ref[2]
    m = mask_ref[0]
    dlist = []
    for t, (i, j, k) in enumerate(_IJK):
        ax = corners_ref[i]
        ay = corners_ref[2 + j]
        az = corners_ref[4 + k]
        d = jnp.sqrt((dx - ax) ** 2 + (dy - ay) ** 2 + (dz - az) ** 2)
        dlist.append(d)
        sidx_ref[t] = jnp.clip(
            ((xc + (i - 3)) * GD + (yc + (j - 3))) * GD + (zc + (k - 3)),
            0, NV - 1)
    for ch in range(CP):
        f = feat_ref[ch] * m
        for t in range(8):
            svals_ref[ch, t] = f / dlist[t]
    for t, (i, j, k) in enumerate(_IJK):
        d8_ref[t] = dlist[4 * i + 2 * j * k] * m
    for t2, (u, v, w) in enumerate(_UVW):
        gidx_ref[t2] = jnp.clip(
            ((xc + (u - 3)) * GD + (yc + (v - 3))) * GD + (zc + (w - 3)),
            0, NV - 1)


def _final_body(g_ref, w_ref, d_ref, b_ref, o_ref):
    y = lax.dot_general(w_ref[...], g_ref[...], (((0,), (0,)), ((), ())),
                        preferred_element_type=jnp.float32)
    y = y + b_ref[...]
    d8 = d_ref[...]
    rows = [jnp.sum(y[c * 8:(c + 1) * 8, :] * d8, axis=0, keepdims=True)
            for c in range(C)]
    o_ref[...] = jnp.concatenate(rows, axis=0) * 0.125


def _sc_body(sidx_hbm, svals_hbm, gidx_hbm, zeros_hbm, out_hbm,
             grid_sh, ib, vb, gib, gvb, sem):
    cid = lax.axis_index("c")
    sid = lax.axis_index("s")
    zbase = pl.multiple_of(sid * ZCHUNK, 8)
    sbase = pl.multiple_of(sid * SPT, 8)
    for r in range(2):
        ch = cid * 2 + r
        zcp = pltpu.async_copy(zeros_hbm, grid_sh.at[pl.ds(zbase, ZCHUNK)],
                               sem)
        pltpu.sync_copy(sidx_hbm.at[pl.ds(sbase, SPT)], ib)
        pltpu.sync_copy(svals_hbm.at[ch, pl.ds(sbase, SPT)], vb)
        zcp.wait()
        plsc.subcore_barrier()
        for j in range(SPT):
            pltpu.sync_copy(vb.at[j], grid_sh.at[ib.at[j]], add=True)
        plsc.subcore_barrier()

        def _gather(g, carry):
            gbase = pl.multiple_of(sid * GPT + g * GCH, 8)
            pltpu.sync_copy(gidx_hbm.at[pl.ds(gbase, GCH)], gib)
            cps = [pltpu.async_copy(grid_sh.at[gib.at[j]], gvb.at[j], sem)
                   for j in range(GCH)]
            for cp in cps:
                cp.wait()
            pltpu.sync_copy(gvb, out_hbm.at[ch, pl.ds(gbase, GCH)])
            return carry

        lax.fori_loop(0, GPT // GCH, _gather, 0)
        plsc.subcore_barrier()


@functools.lru_cache(maxsize=1)
def _sc_splat_gather():
    return functools.partial(
        pl.kernel,
        mesh=plsc.VectorSubcoreMesh(core_axis_name="c", subcore_axis_name="s"),
        out_type=jax.ShapeDtypeStruct((C, GROWS, 128), jnp.float32),
        scratch_types=[
            pltpu.VMEM_SHARED((NVP,), jnp.float32),
            pltpu.VMEM((SPT, 128), jnp.int32),
            pltpu.VMEM((SPT, 128), jnp.float32),
            pltpu.VMEM((GCH, 128), jnp.int32),
            pltpu.VMEM((GCH, 128), jnp.float32),
            pltpu.SemaphoreType.DMA,
        ],
    )(_sc_body)


def _pad_points(a):
    return jnp.concatenate([a, jnp.broadcast_to(a[0], (PP - P,) + a.shape[1:])])


def kernel(data, prev_features, conv_w, conv_b):
    feat = prev_features.reshape(CP, P)
    maxd = jnp.ceil(jnp.max(data, axis=0))
    mind = jnp.floor(jnp.min(data, axis=0))
    s = (maxd - mind) / (N - 9)
    xs = jnp.linspace(mind[0] - s[0] * 4, maxd[0] + s[0] * 4, N)
    ys = jnp.linspace(mind[1] - s[1] * 4, maxd[1] + s[1] * 4, N)
    zs = jnp.linspace(mind[2] - s[2] * 4, maxd[2] + s[2] * 4, N)
    xc = lax.stop_gradient(jnp.floor((data[:, 0] - xs[0]) / s[0])).astype(jnp.int32)
    yc = lax.stop_gradient(jnp.floor((data[:, 1] - ys[0]) / s[1])).astype(jnp.int32)
    zc = lax.stop_gradient(jnp.floor((data[:, 2] - zs[2 - 2]) / s[2])).astype(jnp.int32)
    corners = jnp.stack([
        xs[xc], xs[xc + 1], ys[yc], ys[yc + 1], zs[zc], zs[zc + 1]
    ]).astype(jnp.int32).astype(jnp.float32)

    cells = jnp.stack([xc, yc, zc])
    mask = (jnp.arange(PP) < P).astype(jnp.float32).reshape(1, PR, 128)
    cells_p = _pad_points(cells.T).T.reshape(3, PR, 128)
    corners_p = _pad_points(corners.T).T.reshape(6, PR, 128)
    data_p = _pad_points(data).T.reshape(3, PR, 128)
    feat_p = _pad_points(feat.T).T.reshape(CP, PR, 128)

    sidx, svals, gidx, d8 = pl.pallas_call(
        _prep_body,
        out_shape=[
            jax.ShapeDtypeStruct((8, PR, 128), jnp.int32),
            jax.ShapeDtypeStruct((CP, 8, PR, 128), jnp.float32),
            jax.ShapeDtypeStruct((64, PR, 128), jnp.int32),
            jax.ShapeDtypeStruct((8, PR, 128), jnp.float32),
        ],
    )(cells_p, corners_p, data_p, feat_p, mask)

    pad_rows = SROWS_PAD - SROWS
    G = _sc_splat_gather()(
        jnp.concatenate([sidx.reshape(SROWS, 128),
                         jnp.zeros((pad_rows, 128), jnp.int32)]),
        jnp.concatenate([svals.reshape(CP, SROWS, 128),
                         jnp.zeros((CP, pad_rows, 128), jnp.float32)], axis=1),
        gidx.reshape(GROWS, 128),
        jnp.zeros((ZCHUNK,), jnp.float32),
    )
    Gm = G.reshape(CP * 64, PP)

    wp = jnp.pad(conv_w, ((0, 0), (0, 0), (1, 1), (1, 1), (1, 1)))
    blocks = [wp[:, :, 1 - i:5 - i, 1 - j:5 - j, 1 - k:5 - k]
              for (i, j, k) in _IJK]
    Wm = jnp.stack(blocks, 1).reshape(C, 8, CP, 64)
    Wm = Wm.transpose(2, 3, 0, 1).reshape(CP * 64, C * 8)
    bcol = jnp.repeat(conv_b, 8).reshape(C * 8, 1)
    D8m = d8.reshape(8, PP)

    out = pl.pallas_call(
        _final_body,
        out_shape=jax.ShapeDtypeStruct((C, PP), jnp.float32),
    )(Gm, Wm, D8m, bcol)
    return out[:, :P].reshape(1, C, P, 1, 1)
